# Initial kernel scaffold; baseline (speedup 1.0000x reference)
import jax
import jax.numpy as jnp
from jax.experimental import pallas as pl


def kernel(x, edge_index, W1, b1, W2, b2, W3, b3, Wfc, bfc):
    raise NotImplementedError("write your pallas kernel here")


# speedup vs baseline: 12.0178x; 12.0178x over previous
import jax
import jax.numpy as jnp
from jax import lax
from jax.experimental import pallas as pl
from jax.experimental.pallas import tpu as pltpu
from jax.experimental.pallas import tpu_sc as plsc

NN = 10000
EE = 320000
DD = 128
DH = DD // 2
DG = 16
NC = 2
NS = 16
NW = NC * NS
KCH = 80
CHA = EE // (NS * KCH)
CHD = EE // (NW * KCH)
RB = 200
NBL = NN // RB

_SC_PARAMS = pltpu.CompilerParams(use_tc_tiling_on_sc=False)


def _mesh():
    return plsc.VectorSubcoreMesh(core_axis_name="c", subcore_axis_name="s",
                                  num_cores=NC, num_subcores=NS)


def _sc_agg_body(tab3, srcr, dstr, out3, idx_s, idx_d, rows, stage, acc):
    c = lax.axis_index("c")
    s = lax.axis_index("s")
    pltpu.sync_copy(srcr.at[s], idx_s)
    pltpu.sync_copy(dstr.at[s], idx_d)
    for q in range((NBL + NS - 1) // NS):
        bid = s + q * NS

        @pl.when(bid < NBL)
        def _():
            pltpu.sync_copy(tab3.at[c, pl.ds(bid * RB, RB)], stage)
            pltpu.sync_copy(stage, acc.at[pl.ds(bid * RB, RB)])

    plsc.subcore_barrier()

    def chunk(j, carry):
        pltpu.sync_copy(tab3.at[c].at[idx_s.at[j]], rows)
        pltpu.sync_copy(rows, acc.at[idx_d.at[j]], add=True)
        return carry

    lax.fori_loop(0, CHA, chunk, 0)
    plsc.subcore_barrier()
    for q in range((NBL + NS - 1) // NS):
        bid = s + q * NS

        @pl.when(bid < NBL)
        def _():
            pltpu.sync_copy(acc.at[pl.ds(bid * RB, RB)], stage)
            pltpu.sync_copy(stage, out3.at[c, pl.ds(bid * RB, RB)])


def _make_sc_agg():
    return pl.kernel(
        _sc_agg_body,
        out_type=jax.ShapeDtypeStruct((NC, NN, DH), jnp.float32),
        mesh=_mesh(),
        compiler_params=_SC_PARAMS,
        scratch_types=[
            pltpu.VMEM((CHA, KCH), jnp.int32),
            pltpu.VMEM((CHA, KCH), jnp.int32),
            pltpu.VMEM((KCH, DH), jnp.float32),
            pltpu.VMEM((RB, DH), jnp.float32),
            pltpu.VMEM_SHARED((NN, DH), jnp.float32),
        ],
    )


def _sc_deg_body(ones2, dstr, out3, idx_d, stage, acc):
    c = lax.axis_index("c")
    s = lax.axis_index("s")
    w = c * NS + s
    pltpu.sync_copy(dstr.at[w], idx_d)
    pltpu.sync_copy(ones2, stage)
    for q in range((NBL + NS - 1) // NS):
        bid = s + q * NS

        @pl.when(bid < NBL)
        def _():
            pltpu.sync_copy(stage, acc.at[pl.ds(bid * RB, RB)])

    plsc.subcore_barrier()

    def chunk(j, carry):
        pltpu.sync_copy(stage.at[pl.ds(0, KCH)], acc.at[idx_d.at[j]], add=True)
        return carry

    lax.fori_loop(0, CHD, chunk, 0)
    plsc.subcore_barrier()
    for q in range((NBL + NS - 1) // NS):
        bid = s + q * NS

        @pl.when(bid < NBL)
        def _():
            pltpu.sync_copy(acc.at[pl.ds(bid * RB, RB)], stage)
            pltpu.sync_copy(stage, out3.at[c, pl.ds(bid * RB, RB)])
            pltpu.sync_copy(ones2, stage)


def _make_sc_deg():
    return pl.kernel(
        _sc_deg_body,
        out_type=jax.ShapeDtypeStruct((NC, NN, DG), jnp.float32),
        mesh=_mesh(),
        compiler_params=_SC_PARAMS,
        scratch_types=[
            pltpu.VMEM((CHD, KCH), jnp.int32),
            pltpu.VMEM((RB, DG), jnp.float32),
            pltpu.VMEM_SHARED((NN, DG), jnp.float32),
        ],
    )


def _tc_first_body(x_ref, w_ref, p_ref):
    p_ref[...] = jnp.dot(x_ref[...], w_ref[...],
                         preferred_element_type=jnp.float32)


def _tc_scale_body(deg_ref, p_ref, m_ref, dis_ref):
    deg = deg_ref[0, :, 0:1] + deg_ref[1, :, 0:1] - 1.0
    dis = lax.rsqrt(deg)
    dis_ref[...] = dis
    res = p_ref[...] * dis
    m_ref[0] = res[:, :DH]
    m_ref[1] = res[:, DH:]


def _tc_mid_body(agg_ref, dis_ref, b_ref, w_ref, mout_ref):
    dis = dis_ref[...]
    agg = jnp.concatenate([agg_ref[0], agg_ref[1]], axis=1)
    h = jnp.maximum(agg * dis + b_ref[...], 0.0)
    res = jnp.dot(h, w_ref[...], preferred_element_type=jnp.float32) * dis
    mout_ref[0] = res[:, :DH]
    mout_ref[1] = res[:, DH:]


def _tc_last_body(agg_ref, dis_ref, b_ref, wfc_ref, bfc_ref, out_ref):
    agg = jnp.concatenate([agg_ref[0], agg_ref[1]], axis=1)
    h = jnp.maximum(agg * dis_ref[...] + b_ref[...], 0.0)
    out_ref[...] = jnp.dot(h, wfc_ref[...],
                           preferred_element_type=jnp.float32) + bfc_ref[...]


def _tc(body, out_shapes, *args):
    return pl.pallas_call(body, out_shape=out_shapes)(*args)


def kernel(x, edge_index, W1, b1, W2, b2, W3, b3, Wfc, bfc):
    srcr = edge_index[0].reshape(NS, CHA, KCH)
    dstr = edge_index[1].reshape(NS, CHA, KCH)
    dstr32 = edge_index[1].reshape(NW, CHD, KCH)
    ones2 = jnp.ones((RB, DG), dtype=jnp.float32)
    b1r = b1.reshape(1, -1)
    b2r = b2.reshape(1, -1)
    b3r = b3.reshape(1, -1)
    bfcr = bfc.reshape(1, -1)

    deg3 = _make_sc_deg()(ones2, dstr32)
    p1 = _tc(_tc_first_body, jax.ShapeDtypeStruct((NN, DD), jnp.float32),
             x, W1)
    m1, dis = _tc(_tc_scale_body,
                  (jax.ShapeDtypeStruct((NC, NN, DH), jnp.float32),
                   jax.ShapeDtypeStruct((NN, 1), jnp.float32)),
                  deg3, p1)

    sc_agg = _make_sc_agg()
    agg1 = sc_agg(m1, srcr, dstr)
    m2 = _tc(_tc_mid_body, jax.ShapeDtypeStruct((NC, NN, DH), jnp.float32),
             agg1, dis, b1r, W2)
    agg2 = sc_agg(m2, srcr, dstr)
    m3 = _tc(_tc_mid_body, jax.ShapeDtypeStruct((NC, NN, DH), jnp.float32),
             agg2, dis, b2r, W3)
    agg3 = sc_agg(m3, srcr, dstr)
    out = _tc(_tc_last_body, jax.ShapeDtypeStruct((NN, 1), jnp.float32),
              agg3, dis, b3r, Wfc, bfcr)
    return out
